# baseline (speedup 1.0000x reference)
import jax
import jax.numpy as jnp
from jax.experimental import pallas as pl

N = 65536
C = 1000
H = 512
BLOCK_N = 1024


def _tc_body(x_ref, w_ref, w1_ref, b1_ref, w2_ref, b2_ref, out_ref):
    x = x_ref[...]
    w = jax.nn.relu(w_ref[...])
    m = jnp.max(x, axis=1, keepdims=True)
    s = jnp.sum(jnp.exp(x - m) * w, axis=1, keepdims=True)
    e = m + jnp.log(s)
    h = jax.nn.relu(e * w1_ref[...] + b1_ref[...])
    d = jnp.sum(h * w2_ref[...], axis=1, keepdims=True) + b2_ref[0, 0]
    out_ref[...] = jax.nn.sigmoid(d)


def kernel(cls_logits, energy_score_weights, W1, b1, W2, b2):
    w_row = energy_score_weights.reshape(1, C)
    w1_row = W1.reshape(1, H)
    b1_row = b1.reshape(1, H)
    w2_row = W2.reshape(1, H)
    b2_2d = b2.reshape(1, 1)

    grid = (N // BLOCK_N,)
    out = pl.pallas_call(
        _tc_body,
        grid=grid,
        in_specs=[
            pl.BlockSpec((BLOCK_N, C), lambda i: (i, 0)),
            pl.BlockSpec((1, C), lambda i: (0, 0)),
            pl.BlockSpec((1, H), lambda i: (0, 0)),
            pl.BlockSpec((1, H), lambda i: (0, 0)),
            pl.BlockSpec((1, H), lambda i: (0, 0)),
            pl.BlockSpec((1, 1), lambda i: (0, 0)),
        ],
        out_specs=pl.BlockSpec((BLOCK_N, 1), lambda i: (i, 0)),
        out_shape=jax.ShapeDtypeStruct((N, 1), jnp.float32),
    )(cls_logits, w_row, w1_row, b1_row, w2_row, b2_2d)
    return out


# speedup vs baseline: 1.0709x; 1.0709x over previous
import jax
import jax.numpy as jnp
from jax.experimental import pallas as pl

N = 65536
C = 1000
H = 512
BLOCK_N = 4096


def _tc_body(x_ref, w_ref, w1_ref, b1_ref, w2_ref, b2_ref, out_ref):
    x = x_ref[...]
    w = jax.nn.relu(w_ref[...])
    m = jnp.max(x, axis=1, keepdims=True)
    s = jnp.sum(jnp.exp(x - m) * w, axis=1, keepdims=True)
    e = m + jnp.log(s)
    h = jax.nn.relu(e * w1_ref[...] + b1_ref[...])
    d = jnp.sum(h * w2_ref[...], axis=1, keepdims=True) + b2_ref[0, 0]
    out_ref[...] = jax.nn.sigmoid(d)


def kernel(cls_logits, energy_score_weights, W1, b1, W2, b2):
    w_row = energy_score_weights.reshape(1, C)
    w1_row = W1.reshape(1, H)
    b1_row = b1.reshape(1, H)
    w2_row = W2.reshape(1, H)
    b2_2d = b2.reshape(1, 1)

    grid = (N // BLOCK_N,)
    out = pl.pallas_call(
        _tc_body,
        grid=grid,
        in_specs=[
            pl.BlockSpec((BLOCK_N, C), lambda i: (i, 0)),
            pl.BlockSpec((1, C), lambda i: (0, 0)),
            pl.BlockSpec((1, H), lambda i: (0, 0)),
            pl.BlockSpec((1, H), lambda i: (0, 0)),
            pl.BlockSpec((1, H), lambda i: (0, 0)),
            pl.BlockSpec((1, 1), lambda i: (0, 0)),
        ],
        out_specs=pl.BlockSpec((BLOCK_N, 1), lambda i: (i, 0)),
        out_shape=jax.ShapeDtypeStruct((N, 1), jnp.float32),
    )(cls_logits, w_row, w1_row, b1_row, w2_row, b2_2d)
    return out
